# Initial kernel scaffold; baseline (speedup 1.0000x reference)
import jax
import jax.numpy as jnp
from jax.experimental import pallas as pl


def kernel(dynamic_features, static_features, edge_index, edge_weight, W_ih, W_hh, b_ih, b_hh, W_s, b_s, W_f, b_f, cheb1_W, cheb1_b, cheb2_W, cheb2_b, W_o, b_o):
    raise NotImplementedError("write your pallas kernel here")


# speedup vs baseline: 3.9376x; 3.9376x over previous
import functools

import jax
import jax.numpy as jnp
from jax import lax
from jax.experimental import pallas as pl
from jax.experimental.pallas import tpu as pltpu
from jax.experimental.pallas import tpu_sc as plsc

B, T, N, F_DYN = 2, 12, 10000, 8
F_STA = 16
H = 64
BN = B * N
E = 320000

NC, NS = 2, 16
NW = NC * NS
CHUNK = 128
EPW = -(-(B * E) // (NW * CHUNK)) * CHUNK
E_PAD = EPW * NW
NCHUNK = EPW // CHUNK

RBLK = 2000
DRAIN = 400
NCD = BN // DRAIN


def _encoder_body(x_ref, sta_ref, wih_ref, whh_ref, bg_ref, ws_ref, bs_ref,
                  wf_ref, bf_ref, out_ref):
    wih = wih_ref[...]
    whh = whh_ref[...]
    bg = bg_ref[...]
    h = jnp.zeros((RBLK, H), jnp.float32)
    c = jnp.zeros((RBLK, H), jnp.float32)
    for t in range(T):
        xt = x_ref[0, t]
        g = (jnp.dot(xt, wih, preferred_element_type=jnp.float32)
             + jnp.dot(h, whh, preferred_element_type=jnp.float32) + bg)
        i_g = jax.nn.sigmoid(g[:, 0:H])
        f_g = jax.nn.sigmoid(g[:, H:2 * H])
        g_g = jnp.tanh(g[:, 2 * H:3 * H])
        o_g = jax.nn.sigmoid(g[:, 3 * H:4 * H])
        c = f_g * c + i_g * g_g
        h = o_g * jnp.tanh(c)
    s = jnp.maximum(
        jnp.dot(sta_ref[0], ws_ref[...], preferred_element_type=jnp.float32)
        + bs_ref[...], 0.0)
    wf = wf_ref[...]
    fused = (jnp.dot(h, wf[0:H], preferred_element_type=jnp.float32)
             + jnp.dot(s, wf[H:2 * H], preferred_element_type=jnp.float32)
             + bf_ref[...])
    out_ref[...] = jnp.maximum(fused, 0.0)


def _encoder(dyn, sta, wihT, whhT, bg, wsT, bs, wfT, bf):
    nb = N // RBLK
    return pl.pallas_call(
        _encoder_body,
        grid=(B, nb),
        in_specs=[
            pl.BlockSpec((1, T, RBLK, F_DYN), lambda b, i: (b, 0, i, 0)),
            pl.BlockSpec((1, RBLK, F_STA), lambda b, i: (b, i, 0)),
            pl.BlockSpec((F_DYN, 4 * H), lambda b, i: (0, 0)),
            pl.BlockSpec((H, 4 * H), lambda b, i: (0, 0)),
            pl.BlockSpec((1, 4 * H), lambda b, i: (0, 0)),
            pl.BlockSpec((F_STA, H), lambda b, i: (0, 0)),
            pl.BlockSpec((1, H), lambda b, i: (0, 0)),
            pl.BlockSpec((2 * H, H), lambda b, i: (0, 0)),
            pl.BlockSpec((1, H), lambda b, i: (0, 0)),
        ],
        out_specs=pl.BlockSpec((RBLK, H), lambda b, i: (b * nb + i, 0)),
        out_shape=jax.ShapeDtypeStruct((BN, H), jnp.float32),
    )(dyn, sta, wihT, whhT, bg, wsT, bs, wfT, bf)


def _prop_body(h_hbm, src_hbm, dst_hbm, norm_hbm, out_hbm,
               acc, srcv, dstv, normv, rows, zbuf, sem):
    cid = lax.axis_index("c")
    sid = lax.axis_index("s")
    wid = sid * NC + cid

    def _zb(i, _):
        r = i // 4
        q = i - r * 4
        zbuf[r, pl.ds(q * 16, 16)] = jnp.zeros((16,), jnp.float32)
        return 0
    lax.fori_loop(0, DRAIN * 4, _zb, 0)
    for j in range(-(-NCD // NS)):
        idx = sid + j * NS
        @pl.when(idx < NCD)
        def _():
            pltpu.sync_copy(zbuf, acc.at[pl.ds(idx * DRAIN, DRAIN)])
    plsc.subcore_barrier()

    base = wid * EPW

    def _chunk(ci, _):
        off = base + ci * CHUNK
        pltpu.sync_copy(src_hbm.at[pl.ds(off, CHUNK)], srcv)
        pltpu.sync_copy(dst_hbm.at[pl.ds(off, CHUNK)], dstv)
        pltpu.sync_copy(norm_hbm.at[pl.ds(off, CHUNK)], normv)
        pltpu.async_copy(h_hbm.at[srcv], rows, sem).wait()

        def _group(g, _):
            nvec = normv[pl.ds(g * 16, 16)]
            for l in range(16):
                e = g * 16 + l
                nsp = jnp.broadcast_to(nvec[l], (16,))
                for q in range(H // 16):
                    rows[e, pl.ds(q * 16, 16)] = rows[e, pl.ds(q * 16, 16)] * nsp
            return 0
        lax.fori_loop(0, CHUNK // 16, _group, 0)
        pltpu.sync_copy(rows, acc.at[dstv], add=True)
        return 0
    lax.fori_loop(0, NCHUNK, _chunk, 0)
    plsc.subcore_barrier()

    for j in range(-(-NCD // NS)):
        idx = sid + j * NS
        @pl.when(idx < NCD)
        def _():
            r0 = idx * DRAIN
            pltpu.sync_copy(acc.at[pl.ds(r0, DRAIN)], out_hbm.at[cid, pl.ds(r0, DRAIN)])


@functools.cache
def _prop_kernel():
    return pl.kernel(
        _prop_body,
        out_type=jax.ShapeDtypeStruct((NC, BN, H), jnp.float32),
        mesh=plsc.VectorSubcoreMesh(core_axis_name="c", subcore_axis_name="s"),
        compiler_params=pltpu.CompilerParams(use_tc_tiling_on_sc=False),
        scratch_types=[
            pltpu.VMEM_SHARED((BN, H), jnp.float32),
            pltpu.VMEM((CHUNK,), jnp.int32),
            pltpu.VMEM((CHUNK,), jnp.int32),
            pltpu.VMEM((CHUNK,), jnp.float32),
            pltpu.VMEM((CHUNK, H), jnp.float32),
            pltpu.VMEM((DRAIN, H), jnp.float32),
            pltpu.SemaphoreType.DMA,
        ],
    )


def _prop(h, src, dst, norm):
    return _prop_kernel()(h, src, dst, norm)


def _comb_a_body(p_ref, x_ref, w0_ref, w1_ref, t1_ref, part_ref):
    t1 = p_ref[0] + p_ref[1]
    t1_ref[...] = t1
    part_ref[...] = (jnp.dot(x_ref[...], w0_ref[...], preferred_element_type=jnp.float32)
                     + jnp.dot(t1, w1_ref[...], preferred_element_type=jnp.float32))


def _comb_a(p, x, w0T, w1T):
    nb = BN // RBLK
    return pl.pallas_call(
        _comb_a_body,
        grid=(nb,),
        in_specs=[
            pl.BlockSpec((NC, RBLK, H), lambda i: (0, i, 0)),
            pl.BlockSpec((RBLK, H), lambda i: (i, 0)),
            pl.BlockSpec((H, H), lambda i: (0, 0)),
            pl.BlockSpec((H, H), lambda i: (0, 0)),
        ],
        out_specs=[
            pl.BlockSpec((RBLK, H), lambda i: (i, 0)),
            pl.BlockSpec((RBLK, H), lambda i: (i, 0)),
        ],
        out_shape=[
            jax.ShapeDtypeStruct((BN, H), jnp.float32),
            jax.ShapeDtypeStruct((BN, H), jnp.float32),
        ],
    )(p, x, w0T, w1T)


def _comb_b_body(final, part_ref, p_ref, x_ref, w2_ref, b_ref, wo_ref, bo_ref,
                 y_ref, pred_ref):
    t2 = 2.0 * (p_ref[0] + p_ref[1]) - x_ref[...]
    y = (part_ref[...]
         + jnp.dot(t2, w2_ref[...], preferred_element_type=jnp.float32)
         + b_ref[...])
    if final:
        y_ref[...] = y
        pred_ref[...] = (jnp.dot(y, wo_ref[...], preferred_element_type=jnp.float32)
                         + bo_ref[...])
    else:
        y_ref[...] = jnp.maximum(y, 0.0)
        pred_ref[...] = jnp.zeros((RBLK, 1), jnp.float32)


def _comb_b(part, p, x, w2T, bvec, woT, bo, final):
    nb = BN // RBLK
    body = functools.partial(_comb_b_body, final)
    out_specs = [pl.BlockSpec((RBLK, H), lambda i: (i, 0))]
    out_shape = [jax.ShapeDtypeStruct((BN, H), jnp.float32)]
    out_specs.append(pl.BlockSpec((RBLK, 1), lambda i: (i, 0)))
    out_shape.append(jax.ShapeDtypeStruct((BN, 1), jnp.float32))
    return pl.pallas_call(
        body,
        grid=(nb,),
        in_specs=[
            pl.BlockSpec((RBLK, H), lambda i: (i, 0)),
            pl.BlockSpec((NC, RBLK, H), lambda i: (0, i, 0)),
            pl.BlockSpec((RBLK, H), lambda i: (i, 0)),
            pl.BlockSpec((H, H), lambda i: (0, 0)),
            pl.BlockSpec((1, H), lambda i: (0, 0)),
            pl.BlockSpec((H, 1), lambda i: (0, 0)),
            pl.BlockSpec((1, 1), lambda i: (0, 0)),
        ],
        out_specs=out_specs,
        out_shape=out_shape,
    )(part, p, x, w2T, bvec, woT, bo)


def kernel(dynamic_features, static_features, edge_index, edge_weight,
           W_ih, W_hh, b_ih, b_hh, W_s, b_s, W_f, b_f,
           cheb1_W, cheb1_b, cheb2_W, cheb2_b, W_o, b_o):
    wihT = W_ih.T
    whhT = W_hh.T
    bg = (b_ih + b_hh).reshape(1, 4 * H)
    wsT = W_s.T
    bs = b_s.reshape(1, H)
    wfT = W_f.T
    bf = b_f.reshape(1, H)
    c1 = [cheb1_W[k].T for k in range(3)]
    c2 = [cheb2_W[k].T for k in range(3)]
    b1 = cheb1_b.reshape(1, H)
    b2 = cheb2_b.reshape(1, H)
    woT = W_o.T
    bo = b_o.reshape(1, 1)

    src0 = edge_index[0]
    dst0 = edge_index[1]

    deg = jnp.zeros((N,), jnp.float32).at[src0].add(edge_weight)
    dinv = jnp.where(deg > 0, lax.rsqrt(jnp.where(deg > 0, deg, 1.0)), 0.0)
    norm = -dinv[src0] * edge_weight * dinv[dst0]

    pad = E_PAD - B * E
    src_full = jnp.concatenate([src0, src0 + N, jnp.zeros((pad,), jnp.int32)])
    dst_full = jnp.concatenate([dst0, dst0 + N, jnp.zeros((pad,), jnp.int32)])
    norm_full = jnp.concatenate([norm, norm, jnp.zeros((pad,), jnp.float32)])

    x0 = _encoder(dynamic_features, static_features,
                  wihT, whhT, bg, wsT, bs, wfT, bf)

    p1 = _prop(x0, src_full, dst_full, norm_full)
    t1, part1 = _comb_a(p1, x0, c1[0], c1[1])
    p2 = _prop(t1, src_full, dst_full, norm_full)
    y1, _ = _comb_b(part1, p2, x0, c1[2], b1, woT, bo, final=False)

    q1 = _prop(y1, src_full, dst_full, norm_full)
    u1, part2 = _comb_a(q1, y1, c2[0], c2[1])
    q2 = _prop(u1, src_full, dst_full, norm_full)
    _, pred = _comb_b(part2, q2, y1, c2[2], b2, woT, bo, final=True)

    return pred.reshape(B, N)


# speedup vs baseline: 4.6459x; 1.1799x over previous
import functools

import jax
import jax.numpy as jnp
from jax import lax
from jax.experimental import pallas as pl
from jax.experimental.pallas import tpu as pltpu
from jax.experimental.pallas import tpu_sc as plsc

B, T, N, F_DYN = 2, 12, 10000, 8
F_STA = 16
H = 64
BN = B * N
E = 320000

NC, NS = 2, 16
NW = NC * NS
CHUNK = 128
RING = 4
NCHUNK = RING * (-(-(B * E) // (NW * CHUNK * RING)))
EPW = NCHUNK * CHUNK
E_PAD = EPW * NW
NCHT = E_PAD // CHUNK

RBLK = 2000
DRAIN = 400
NCD = BN // DRAIN
ZROWS = 80
NZC = BN // ZROWS


def _encoder_body(x_ref, sta_ref, wih_ref, whh_ref, bg_ref, ws_ref, bs_ref,
                  wf_ref, bf_ref, out_ref):
    wih = wih_ref[...]
    whh = whh_ref[...]
    bg = bg_ref[...]
    h = jnp.zeros((RBLK, H), jnp.float32)
    c = jnp.zeros((RBLK, H), jnp.float32)
    for t in range(T):
        xt = x_ref[0, t]
        g = (jnp.dot(xt, wih, preferred_element_type=jnp.float32)
             + jnp.dot(h, whh, preferred_element_type=jnp.float32) + bg)
        i_g = jax.nn.sigmoid(g[:, 0:H])
        f_g = jax.nn.sigmoid(g[:, H:2 * H])
        g_g = jnp.tanh(g[:, 2 * H:3 * H])
        o_g = jax.nn.sigmoid(g[:, 3 * H:4 * H])
        c = f_g * c + i_g * g_g
        h = o_g * jnp.tanh(c)
    s = jnp.maximum(
        jnp.dot(sta_ref[0], ws_ref[...], preferred_element_type=jnp.float32)
        + bs_ref[...], 0.0)
    wf = wf_ref[...]
    fused = (jnp.dot(h, wf[0:H], preferred_element_type=jnp.float32)
             + jnp.dot(s, wf[H:2 * H], preferred_element_type=jnp.float32)
             + bf_ref[...])
    out_ref[...] = jnp.maximum(fused, 0.0)


def _encoder(dyn, sta, wihT, whhT, bg, wsT, bs, wfT, bf):
    nb = N // RBLK
    return pl.pallas_call(
        _encoder_body,
        grid=(B, nb),
        in_specs=[
            pl.BlockSpec((1, T, RBLK, F_DYN), lambda b, i: (b, 0, i, 0)),
            pl.BlockSpec((1, RBLK, F_STA), lambda b, i: (b, i, 0)),
            pl.BlockSpec((F_DYN, 4 * H), lambda b, i: (0, 0)),
            pl.BlockSpec((H, 4 * H), lambda b, i: (0, 0)),
            pl.BlockSpec((1, 4 * H), lambda b, i: (0, 0)),
            pl.BlockSpec((F_STA, H), lambda b, i: (0, 0)),
            pl.BlockSpec((1, H), lambda b, i: (0, 0)),
            pl.BlockSpec((2 * H, H), lambda b, i: (0, 0)),
            pl.BlockSpec((1, H), lambda b, i: (0, 0)),
        ],
        out_specs=pl.BlockSpec((RBLK, H), lambda b, i: (b * nb + i, 0)),
        out_shape=jax.ShapeDtypeStruct((BN, H), jnp.float32),
    )(dyn, sta, wihT, whhT, bg, wsT, bs, wfT, bf)


def _mul_norm(nbuf, rows, s):
    def _group(g, _):
        nvec = nbuf[s, pl.ds(g * 16, 16)]
        for l in range(16):
            e = g * 16 + l
            nsp = jnp.broadcast_to(nvec[l], (16,))
            for q in range(H // 16):
                rows[s, e, pl.ds(q * 16, 16)] = rows[s, e, pl.ds(q * 16, 16)] * nsp
        return 0
    lax.fori_loop(0, CHUNK // 16, _group, 0)


def _prop_body(h_hbm, ep_hbm, nf_hbm, out_hbm, acc, ebuf, nbuf, rows, zbuf,
               esem, gsem, ssem):
    cid = lax.axis_index("c")
    sid = lax.axis_index("s")
    wid = sid * NC + cid
    base = wid * NCHUNK

    def _edge(ci, s):
        pltpu.async_copy(ep_hbm.at[base + ci], ebuf.at[s], esem.at[s])
        pltpu.async_copy(nf_hbm.at[base + ci], nbuf.at[s], esem.at[s])

    def _wait_edge(s):
        pltpu.make_async_copy(ep_hbm.at[0], ebuf.at[s], esem.at[s]).wait()
        pltpu.make_async_copy(nf_hbm.at[0], nbuf.at[s], esem.at[s]).wait()

    def _gather(ci_unused, s):
        pltpu.async_copy(h_hbm.at[ebuf.at[s, 0]], rows.at[s], gsem.at[s])

    def _wait_gather(s):
        pltpu.make_async_copy(h_hbm.at[ebuf.at[0, 0]], rows.at[s], gsem.at[s]).wait()

    def _scatter(s):
        pltpu.async_copy(rows.at[s], acc.at[ebuf.at[s, 1]], ssem.at[s], add=True)

    def _wait_scatter(s):
        pltpu.make_async_copy(rows.at[s], acc.at[ebuf.at[0, 1]], ssem.at[s]).wait()

    def _zb(i, _):
        r = i // 4
        q = i - r * 4
        zbuf[r, pl.ds(q * 16, 16)] = jnp.zeros((16,), jnp.float32)
        return 0
    lax.fori_loop(0, ZROWS * 4, _zb, 0)
    for j in range(-(-NZC // NS)):
        idx = sid + j * NS
        @pl.when(idx < NZC)
        def _():
            pltpu.sync_copy(zbuf, acc.at[pl.ds(idx * ZROWS, ZROWS)])
    plsc.subcore_barrier()

    _edge(0, 0)
    _edge(1, 1)
    _wait_edge(0)
    _gather(0, 0)

    def _iter(i, _):
        for s in range(RING):
            ci = RING * i + s
            s1 = (s + 1) % RING
            s2 = (s + 2) % RING
            @pl.when(ci + 2 < NCHUNK)
            def _():
                @pl.when(ci >= 2)
                def _():
                    _wait_scatter(s2)
                _edge(ci + 2, s2)
            @pl.when(ci + 1 < NCHUNK)
            def _():
                _wait_edge(s1)
                _gather(ci + 1, s1)
            _wait_gather(s)
            _mul_norm(nbuf, rows, s)
            _scatter(s)
        return 0
    lax.fori_loop(0, NCHUNK // RING, _iter, 0)
    for s in range(RING):
        _wait_scatter(s)
    plsc.subcore_barrier()

    for j in range(-(-NCD // NS)):
        idx = sid + j * NS
        @pl.when(idx < NCD)
        def _():
            r0 = idx * DRAIN
            pltpu.sync_copy(acc.at[pl.ds(r0, DRAIN)], out_hbm.at[cid, pl.ds(r0, DRAIN)])


@functools.cache
def _prop_kernel():
    return pl.kernel(
        _prop_body,
        out_type=jax.ShapeDtypeStruct((NC, BN, H), jnp.float32),
        mesh=plsc.VectorSubcoreMesh(core_axis_name="c", subcore_axis_name="s"),
        compiler_params=pltpu.CompilerParams(use_tc_tiling_on_sc=False),
        scratch_types=[
            pltpu.VMEM_SHARED((BN, H), jnp.float32),
            pltpu.VMEM((RING, 2, CHUNK), jnp.int32),
            pltpu.VMEM((RING, CHUNK), jnp.float32),
            pltpu.VMEM((RING, CHUNK, H), jnp.float32),
            pltpu.VMEM((ZROWS, H), jnp.float32),
            pltpu.SemaphoreType.DMA((RING,)),
            pltpu.SemaphoreType.DMA((RING,)),
            pltpu.SemaphoreType.DMA((RING,)),
        ],
    )


def _prop(h, epack, normf):
    return _prop_kernel()(h, epack, normf)


def _comb_a_body(p_ref, x_ref, w0_ref, w1_ref, t1_ref, part_ref):
    t1 = p_ref[0] + p_ref[1]
    t1_ref[...] = t1
    part_ref[...] = (jnp.dot(x_ref[...], w0_ref[...], preferred_element_type=jnp.float32)
                     + jnp.dot(t1, w1_ref[...], preferred_element_type=jnp.float32))


def _comb_a(p, x, w0T, w1T):
    nb = BN // RBLK
    return pl.pallas_call(
        _comb_a_body,
        grid=(nb,),
        in_specs=[
            pl.BlockSpec((NC, RBLK, H), lambda i: (0, i, 0)),
            pl.BlockSpec((RBLK, H), lambda i: (i, 0)),
            pl.BlockSpec((H, H), lambda i: (0, 0)),
            pl.BlockSpec((H, H), lambda i: (0, 0)),
        ],
        out_specs=[
            pl.BlockSpec((RBLK, H), lambda i: (i, 0)),
            pl.BlockSpec((RBLK, H), lambda i: (i, 0)),
        ],
        out_shape=[
            jax.ShapeDtypeStruct((BN, H), jnp.float32),
            jax.ShapeDtypeStruct((BN, H), jnp.float32),
        ],
    )(p, x, w0T, w1T)


def _comb_b_body(final, part_ref, p_ref, x_ref, w2_ref, b_ref, wo_ref, bo_ref,
                 y_ref, pred_ref):
    t2 = 2.0 * (p_ref[0] + p_ref[1]) - x_ref[...]
    y = (part_ref[...]
         + jnp.dot(t2, w2_ref[...], preferred_element_type=jnp.float32)
         + b_ref[...])
    if final:
        y_ref[...] = y
        pred_ref[...] = (jnp.dot(y, wo_ref[...], preferred_element_type=jnp.float32)
                         + bo_ref[...])
    else:
        y_ref[...] = jnp.maximum(y, 0.0)
        pred_ref[...] = jnp.zeros((RBLK, 1), jnp.float32)


def _comb_b(part, p, x, w2T, bvec, woT, bo, final):
    nb = BN // RBLK
    body = functools.partial(_comb_b_body, final)
    out_specs = [pl.BlockSpec((RBLK, H), lambda i: (i, 0))]
    out_shape = [jax.ShapeDtypeStruct((BN, H), jnp.float32)]
    out_specs.append(pl.BlockSpec((RBLK, 1), lambda i: (i, 0)))
    out_shape.append(jax.ShapeDtypeStruct((BN, 1), jnp.float32))
    return pl.pallas_call(
        body,
        grid=(nb,),
        in_specs=[
            pl.BlockSpec((RBLK, H), lambda i: (i, 0)),
            pl.BlockSpec((NC, RBLK, H), lambda i: (0, i, 0)),
            pl.BlockSpec((RBLK, H), lambda i: (i, 0)),
            pl.BlockSpec((H, H), lambda i: (0, 0)),
            pl.BlockSpec((1, H), lambda i: (0, 0)),
            pl.BlockSpec((H, 1), lambda i: (0, 0)),
            pl.BlockSpec((1, 1), lambda i: (0, 0)),
        ],
        out_specs=out_specs,
        out_shape=out_shape,
    )(part, p, x, w2T, bvec, woT, bo)


def kernel(dynamic_features, static_features, edge_index, edge_weight,
           W_ih, W_hh, b_ih, b_hh, W_s, b_s, W_f, b_f,
           cheb1_W, cheb1_b, cheb2_W, cheb2_b, W_o, b_o):
    wihT = W_ih.T
    whhT = W_hh.T
    bg = (b_ih + b_hh).reshape(1, 4 * H)
    wsT = W_s.T
    bs = b_s.reshape(1, H)
    wfT = W_f.T
    bf = b_f.reshape(1, H)
    c1 = [cheb1_W[k].T for k in range(3)]
    c2 = [cheb2_W[k].T for k in range(3)]
    b1 = cheb1_b.reshape(1, H)
    b2 = cheb2_b.reshape(1, H)
    woT = W_o.T
    bo = b_o.reshape(1, 1)

    src0 = edge_index[0]
    dst0 = edge_index[1]

    deg = jnp.zeros((N,), jnp.float32).at[src0].add(edge_weight)
    dinv = jnp.where(deg > 0, lax.rsqrt(jnp.where(deg > 0, deg, 1.0)), 0.0)
    norm = -dinv[src0] * edge_weight * dinv[dst0]

    pad = E_PAD - B * E
    src_full = jnp.concatenate([src0, src0 + N, jnp.zeros((pad,), jnp.int32)])
    dst_full = jnp.concatenate([dst0, dst0 + N, jnp.zeros((pad,), jnp.int32)])
    norm_full = jnp.concatenate([norm, norm, jnp.zeros((pad,), jnp.float32)])
    epack = jnp.stack([
        src_full.reshape(NCHT, CHUNK),
        dst_full.reshape(NCHT, CHUNK),
    ], axis=1)
    normf = norm_full.reshape(NCHT, CHUNK)

    x0 = _encoder(dynamic_features, static_features,
                  wihT, whhT, bg, wsT, bs, wfT, bf)

    p1 = _prop(x0, epack, normf)
    t1, part1 = _comb_a(p1, x0, c1[0], c1[1])
    p2 = _prop(t1, epack, normf)
    y1, _ = _comb_b(part1, p2, x0, c1[2], b1, woT, bo, final=False)

    q1 = _prop(y1, epack, normf)
    u1, part2 = _comb_a(q1, y1, c2[0], c2[1])
    q2 = _prop(u1, epack, normf)
    _, pred = _comb_b(part2, q2, y1, c2[2], b2, woT, bo, final=True)

    return pred.reshape(B, N)


# speedup vs baseline: 11.0488x; 2.3782x over previous
import functools

import jax
import jax.numpy as jnp
from jax import lax
from jax.experimental import pallas as pl
from jax.experimental.pallas import tpu as pltpu
from jax.experimental.pallas import tpu_sc as plsc

B, T, N, F_DYN = 2, 12, 10000, 8
F_STA = 16
H = 64
BN = B * N
E = 320000

NC, NS = 2, 16
NW = NC * NS
CHUNK = 128
RING = 4
NCHUNK = RING * (-(-(B * E) // (NW * CHUNK * RING)))
EPW = NCHUNK * CHUNK
E_PAD = EPW * NW
NCHT = E_PAD // CHUNK

RBLK = 2000
DRAIN = 400
NCD = BN // DRAIN
ZROWS = 80
NZC = BN // ZROWS


def _encoder_body(x_ref, sta_ref, wih_ref, whh_ref, bg_ref, ws_ref, bs_ref,
                  wf_ref, bf_ref, dv_ref, out_ref, outp_ref):
    wih = wih_ref[...]
    whh = whh_ref[...]
    bg = bg_ref[...]
    h = jnp.zeros((RBLK, H), jnp.float32)
    c = jnp.zeros((RBLK, H), jnp.float32)
    for t in range(T):
        xt = x_ref[0, t]
        g = (jnp.dot(xt, wih, preferred_element_type=jnp.float32)
             + jnp.dot(h, whh, preferred_element_type=jnp.float32) + bg)
        i_g = jax.nn.sigmoid(g[:, 0:H])
        f_g = jax.nn.sigmoid(g[:, H:2 * H])
        g_g = jnp.tanh(g[:, 2 * H:3 * H])
        o_g = jax.nn.sigmoid(g[:, 3 * H:4 * H])
        c = f_g * c + i_g * g_g
        h = o_g * jnp.tanh(c)
    s = jnp.maximum(
        jnp.dot(sta_ref[0], ws_ref[...], preferred_element_type=jnp.float32)
        + bs_ref[...], 0.0)
    wf = wf_ref[...]
    fused = (jnp.dot(h, wf[0:H], preferred_element_type=jnp.float32)
             + jnp.dot(s, wf[H:2 * H], preferred_element_type=jnp.float32)
             + bf_ref[...])
    x0 = jnp.maximum(fused, 0.0)
    out_ref[...] = x0
    outp_ref[...] = dv_ref[...] * x0


def _encoder(dyn, sta, wihT, whhT, bg, wsT, bs, wfT, bf, dv):
    nb = N // RBLK
    return pl.pallas_call(
        _encoder_body,
        grid=(B, nb),
        in_specs=[
            pl.BlockSpec((1, T, RBLK, F_DYN), lambda b, i: (b, 0, i, 0)),
            pl.BlockSpec((1, RBLK, F_STA), lambda b, i: (b, i, 0)),
            pl.BlockSpec((F_DYN, 4 * H), lambda b, i: (0, 0)),
            pl.BlockSpec((H, 4 * H), lambda b, i: (0, 0)),
            pl.BlockSpec((1, 4 * H), lambda b, i: (0, 0)),
            pl.BlockSpec((F_STA, H), lambda b, i: (0, 0)),
            pl.BlockSpec((1, H), lambda b, i: (0, 0)),
            pl.BlockSpec((2 * H, H), lambda b, i: (0, 0)),
            pl.BlockSpec((1, H), lambda b, i: (0, 0)),
            pl.BlockSpec((RBLK, 1), lambda b, i: (b * nb + i, 0)),
        ],
        out_specs=[
            pl.BlockSpec((RBLK, H), lambda b, i: (b * nb + i, 0)),
            pl.BlockSpec((RBLK, H), lambda b, i: (b * nb + i, 0)),
        ],
        out_shape=[
            jax.ShapeDtypeStruct((BN, H), jnp.float32),
            jax.ShapeDtypeStruct((BN, H), jnp.float32),
        ],
    )(dyn, sta, wihT, whhT, bg, wsT, bs, wfT, bf, dv)


def _mul_norm(nbuf, rows, s):
    def _group(g, _):
        nvec = nbuf[s, pl.ds(g * 16, 16)]
        for l in range(16):
            e = g * 16 + l
            nsp = jnp.broadcast_to(nvec[l], (16,))
            for q in range(H // 16):
                rows[s, e, pl.ds(q * 16, 16)] = rows[s, e, pl.ds(q * 16, 16)] * nsp
        return 0
    lax.fori_loop(0, CHUNK // 16, _group, 0)


def _prop_body(h_hbm, ep_hbm, nf_hbm, out_hbm, acc, ebuf, nbuf, rows, zbuf,
               esem, gsem, ssem):
    cid = lax.axis_index("c")
    sid = lax.axis_index("s")
    wid = sid * NC + cid
    base = wid * NCHUNK

    def _edge(ci, s):
        pltpu.async_copy(ep_hbm.at[base + ci], ebuf.at[s], esem.at[s])
        pltpu.async_copy(nf_hbm.at[base + ci], nbuf.at[s], esem.at[s])

    def _wait_edge(s):
        pltpu.make_async_copy(ep_hbm.at[0], ebuf.at[s], esem.at[s]).wait()
        pltpu.make_async_copy(nf_hbm.at[0], nbuf.at[s], esem.at[s]).wait()

    def _gather(ci_unused, s):
        pltpu.async_copy(h_hbm.at[ebuf.at[s, 0]], rows.at[s], gsem.at[s])

    def _wait_gather(s):
        pltpu.make_async_copy(h_hbm.at[ebuf.at[0, 0]], rows.at[s], gsem.at[s]).wait()

    def _scatter(s):
        pltpu.async_copy(rows.at[s], acc.at[ebuf.at[s, 1]], ssem.at[s], add=True)

    def _wait_scatter(s):
        pltpu.make_async_copy(rows.at[s], acc.at[ebuf.at[0, 1]], ssem.at[s]).wait()

    def _zb(i, _):
        r = i // 4
        q = i - r * 4
        zbuf[r, pl.ds(q * 16, 16)] = jnp.zeros((16,), jnp.float32)
        return 0
    lax.fori_loop(0, ZROWS * 4, _zb, 0)
    for j in range(-(-NZC // NS)):
        idx = sid + j * NS
        @pl.when(idx < NZC)
        def _():
            pltpu.sync_copy(zbuf, acc.at[pl.ds(idx * ZROWS, ZROWS)])
    plsc.subcore_barrier()

    _edge(0, 0)
    _edge(1, 1)
    _wait_edge(0)
    _gather(0, 0)

    def _iter(i, _):
        for s in range(RING):
            ci = RING * i + s
            s1 = (s + 1) % RING
            s2 = (s + 2) % RING
            @pl.when(ci + 2 < NCHUNK)
            def _():
                @pl.when(ci >= 2)
                def _():
                    _wait_scatter(s2)
                _edge(ci + 2, s2)
            @pl.when(ci + 1 < NCHUNK)
            def _():
                _wait_edge(s1)
                _gather(ci + 1, s1)
            _wait_gather(s)
            _mul_norm(nbuf, rows, s)
            _scatter(s)
        return 0
    lax.fori_loop(0, NCHUNK // RING, _iter, 0)
    for s in range(RING):
        _wait_scatter(s)
    plsc.subcore_barrier()

    for j in range(-(-NCD // NS)):
        idx = sid + j * NS
        @pl.when(idx < NCD)
        def _():
            r0 = idx * DRAIN
            pltpu.sync_copy(acc.at[pl.ds(r0, DRAIN)], out_hbm.at[cid, pl.ds(r0, DRAIN)])


@functools.cache
def _prop_kernel():
    return pl.kernel(
        _prop_body,
        out_type=jax.ShapeDtypeStruct((NC, BN, H), jnp.float32),
        mesh=plsc.VectorSubcoreMesh(core_axis_name="c", subcore_axis_name="s"),
        compiler_params=pltpu.CompilerParams(use_tc_tiling_on_sc=False),
        scratch_types=[
            pltpu.VMEM_SHARED((BN, H), jnp.float32),
            pltpu.VMEM((RING, 2, CHUNK), jnp.int32),
            pltpu.VMEM((RING, CHUNK), jnp.float32),
            pltpu.VMEM((RING, CHUNK, H), jnp.float32),
            pltpu.VMEM((ZROWS, H), jnp.float32),
            pltpu.SemaphoreType.DMA((RING,)),
            pltpu.SemaphoreType.DMA((RING,)),
            pltpu.SemaphoreType.DMA((RING,)),
        ],
    )


def _prop(h, epack, normf):
    return _prop_kernel()(h, epack, normf)


def _comb_a_body(p_ref, x_ref, w0_ref, w1_ref, dv_ref, part_ref, t1p_ref):
    dv = dv_ref[...]
    t1 = dv * (p_ref[0] + p_ref[1])
    t1p_ref[...] = dv * t1
    part_ref[...] = (jnp.dot(x_ref[...], w0_ref[...], preferred_element_type=jnp.float32)
                     + jnp.dot(t1, w1_ref[...], preferred_element_type=jnp.float32))


def _comb_a(p, x, w0T, w1T, dv):
    nb = BN // RBLK
    return pl.pallas_call(
        _comb_a_body,
        grid=(nb,),
        in_specs=[
            pl.BlockSpec((NC, RBLK, H), lambda i: (0, i, 0)),
            pl.BlockSpec((RBLK, H), lambda i: (i, 0)),
            pl.BlockSpec((H, H), lambda i: (0, 0)),
            pl.BlockSpec((H, H), lambda i: (0, 0)),
            pl.BlockSpec((RBLK, 1), lambda i: (i, 0)),
        ],
        out_specs=[
            pl.BlockSpec((RBLK, H), lambda i: (i, 0)),
            pl.BlockSpec((RBLK, H), lambda i: (i, 0)),
        ],
        out_shape=[
            jax.ShapeDtypeStruct((BN, H), jnp.float32),
            jax.ShapeDtypeStruct((BN, H), jnp.float32),
        ],
    )(p, x, w0T, w1T, dv)


def _comb_b_body(final, part_ref, p_ref, x_ref, w2_ref, b_ref, wo_ref, bo_ref,
                 dv_ref, y_ref, yp_ref):
    dv = dv_ref[...]
    t2 = 2.0 * dv * (p_ref[0] + p_ref[1]) - x_ref[...]
    y = (part_ref[...]
         + jnp.dot(t2, w2_ref[...], preferred_element_type=jnp.float32)
         + b_ref[...])
    if final:
        y_ref[...] = y
        yp_ref[...] = (jnp.dot(y, wo_ref[...], preferred_element_type=jnp.float32)
                       + bo_ref[...])
    else:
        y = jnp.maximum(y, 0.0)
        y_ref[...] = y
        yp_ref[...] = dv * y


def _comb_b(part, p, x, w2T, bvec, woT, bo, dv, final):
    nb = BN // RBLK
    body = functools.partial(_comb_b_body, final)
    out_specs = [pl.BlockSpec((RBLK, H), lambda i: (i, 0))]
    out_shape = [jax.ShapeDtypeStruct((BN, H), jnp.float32)]
    if final:
        out_specs.append(pl.BlockSpec((RBLK, 1), lambda i: (i, 0)))
        out_shape.append(jax.ShapeDtypeStruct((BN, 1), jnp.float32))
    else:
        out_specs.append(pl.BlockSpec((RBLK, H), lambda i: (i, 0)))
        out_shape.append(jax.ShapeDtypeStruct((BN, H), jnp.float32))
    return pl.pallas_call(
        body,
        grid=(nb,),
        in_specs=[
            pl.BlockSpec((RBLK, H), lambda i: (i, 0)),
            pl.BlockSpec((NC, RBLK, H), lambda i: (0, i, 0)),
            pl.BlockSpec((RBLK, H), lambda i: (i, 0)),
            pl.BlockSpec((H, H), lambda i: (0, 0)),
            pl.BlockSpec((1, H), lambda i: (0, 0)),
            pl.BlockSpec((H, 1), lambda i: (0, 0)),
            pl.BlockSpec((1, 1), lambda i: (0, 0)),
            pl.BlockSpec((RBLK, 1), lambda i: (i, 0)),
        ],
        out_specs=out_specs,
        out_shape=out_shape,
    )(part, p, x, w2T, bvec, woT, bo, dv)


def kernel(dynamic_features, static_features, edge_index, edge_weight,
           W_ih, W_hh, b_ih, b_hh, W_s, b_s, W_f, b_f,
           cheb1_W, cheb1_b, cheb2_W, cheb2_b, W_o, b_o):
    wihT = W_ih.T
    whhT = W_hh.T
    bg = (b_ih + b_hh).reshape(1, 4 * H)
    wsT = W_s.T
    bs = b_s.reshape(1, H)
    wfT = W_f.T
    bf = b_f.reshape(1, H)
    c1 = [cheb1_W[k].T for k in range(3)]
    c2 = [cheb2_W[k].T for k in range(3)]
    b1 = cheb1_b.reshape(1, H)
    b2 = cheb2_b.reshape(1, H)
    woT = W_o.T
    bo = b_o.reshape(1, 1)

    src0 = edge_index[0]
    dst0 = edge_index[1]

    deg = jnp.zeros((N,), jnp.float32).at[src0].add(edge_weight)
    dinv = jnp.where(deg > 0, lax.rsqrt(jnp.where(deg > 0, deg, 1.0)), 0.0)
    dv = jnp.concatenate([dinv, dinv]).reshape(BN, 1)

    pad = E_PAD - B * E
    src_full = jnp.concatenate([src0, src0 + N, jnp.zeros((pad,), jnp.int32)])
    dst_full = jnp.concatenate([dst0, dst0 + N, jnp.zeros((pad,), jnp.int32)])
    ew_full = jnp.concatenate([-edge_weight, -edge_weight,
                               jnp.zeros((pad,), jnp.float32)])
    epack = jnp.stack([
        src_full.reshape(NCHT, CHUNK),
        dst_full.reshape(NCHT, CHUNK),
    ], axis=1)
    normf = ew_full.reshape(NCHT, CHUNK)

    x0, x0p = _encoder(dynamic_features, static_features,
                       wihT, whhT, bg, wsT, bs, wfT, bf, dv)

    p1 = _prop(x0p, epack, normf)
    part1, t1p = _comb_a(p1, x0, c1[0], c1[1], dv)
    p2 = _prop(t1p, epack, normf)
    y1, y1p = _comb_b(part1, p2, x0, c1[2], b1, woT, bo, dv, final=False)

    q1 = _prop(y1p, epack, normf)
    part2, u1p = _comb_a(q1, y1, c2[0], c2[1], dv)
    q2 = _prop(u1p, epack, normf)
    _, pred = _comb_b(part2, q2, y1, c2[2], b2, woT, bo, dv, final=True)

    return pred.reshape(B, N)


# speedup vs baseline: 12.0084x; 1.0869x over previous
import functools

import jax
import jax.numpy as jnp
from jax import lax
from jax.experimental import pallas as pl
from jax.experimental.pallas import tpu as pltpu
from jax.experimental.pallas import tpu_sc as plsc

B, T, N, F_DYN = 2, 12, 10000, 8
F_STA = 16
H = 64
BN = B * N
E = 320000

NC, NS = 2, 16
NW = NC * NS
CHUNK = 128
RING = 4
NCHUNK = RING * (-(-(B * E) // (NW * CHUNK * RING)))
CA = 212
CB = 2 * NCHUNK - CA
EPW = NCHUNK * CHUNK
E_PAD = EPW * NW
NCHT = E_PAD // CHUNK

RBLK = 2000
DRAIN = 400
NCD = BN // DRAIN
ZROWS = 80
NZC = BN // ZROWS


def _encoder_body(x_ref, sta_ref, wih_ref, whh_ref, bg_ref, ws_ref, bs_ref,
                  wf_ref, bf_ref, dv_ref, out_ref, outp_ref):
    wih = wih_ref[...]
    whh = whh_ref[...]
    bg = bg_ref[...]
    h = jnp.zeros((RBLK, H), jnp.float32)
    c = jnp.zeros((RBLK, H), jnp.float32)
    for t in range(T):
        xt = x_ref[0, t]
        g = (jnp.dot(xt, wih, preferred_element_type=jnp.float32)
             + jnp.dot(h, whh, preferred_element_type=jnp.float32) + bg)
        i_g = jax.nn.sigmoid(g[:, 0:H])
        f_g = jax.nn.sigmoid(g[:, H:2 * H])
        g_g = jnp.tanh(g[:, 2 * H:3 * H])
        o_g = jax.nn.sigmoid(g[:, 3 * H:4 * H])
        c = f_g * c + i_g * g_g
        h = o_g * jnp.tanh(c)
    s = jnp.maximum(
        jnp.dot(sta_ref[0], ws_ref[...], preferred_element_type=jnp.float32)
        + bs_ref[...], 0.0)
    wf = wf_ref[...]
    fused = (jnp.dot(h, wf[0:H], preferred_element_type=jnp.float32)
             + jnp.dot(s, wf[H:2 * H], preferred_element_type=jnp.float32)
             + bf_ref[...])
    x0 = jnp.maximum(fused, 0.0)
    out_ref[...] = x0
    outp_ref[...] = dv_ref[...] * x0


def _encoder(dyn, sta, wihT, whhT, bg, wsT, bs, wfT, bf, dv):
    nb = N // RBLK
    return pl.pallas_call(
        _encoder_body,
        grid=(B, nb),
        in_specs=[
            pl.BlockSpec((1, T, RBLK, F_DYN), lambda b, i: (b, 0, i, 0)),
            pl.BlockSpec((1, RBLK, F_STA), lambda b, i: (b, i, 0)),
            pl.BlockSpec((F_DYN, 4 * H), lambda b, i: (0, 0)),
            pl.BlockSpec((H, 4 * H), lambda b, i: (0, 0)),
            pl.BlockSpec((1, 4 * H), lambda b, i: (0, 0)),
            pl.BlockSpec((F_STA, H), lambda b, i: (0, 0)),
            pl.BlockSpec((1, H), lambda b, i: (0, 0)),
            pl.BlockSpec((2 * H, H), lambda b, i: (0, 0)),
            pl.BlockSpec((1, H), lambda b, i: (0, 0)),
            pl.BlockSpec((RBLK, 1), lambda b, i: (b * nb + i, 0)),
        ],
        out_specs=[
            pl.BlockSpec((RBLK, H), lambda b, i: (b * nb + i, 0)),
            pl.BlockSpec((RBLK, H), lambda b, i: (b * nb + i, 0)),
        ],
        out_shape=[
            jax.ShapeDtypeStruct((BN, H), jnp.float32),
            jax.ShapeDtypeStruct((BN, H), jnp.float32),
        ],
    )(dyn, sta, wihT, whhT, bg, wsT, bs, wfT, bf, dv)


def _mul_norm(nbuf, rows, s):
    def _group(g, _):
        nvec = nbuf[s, pl.ds(g * 16, 16)]
        for l in range(16):
            e = g * 16 + l
            nsp = jnp.broadcast_to(nvec[l], (16,))
            for q in range(H // 16):
                rows[s, e, pl.ds(q * 16, 16)] = rows[s, e, pl.ds(q * 16, 16)] * nsp
        return 0
    lax.fori_loop(0, CHUNK // 16, _group, 0)


def _prop_body(h_hbm, ep_hbm, nf_hbm, out_hbm, acc, ebuf, nbuf, rows, zbuf,
               esem, gsem, ssem):
    cid = lax.axis_index("c")
    sid = lax.axis_index("s")
    myn = jnp.where(cid == 0, CA, CB)
    base = jnp.where(cid == 0, sid * CA, NS * CA + sid * CB)

    def _edge(ci, s):
        pltpu.async_copy(ep_hbm.at[base + ci], ebuf.at[s], esem.at[s])
        pltpu.async_copy(nf_hbm.at[base + ci], nbuf.at[s], esem.at[s])

    def _wait_edge(s):
        pltpu.make_async_copy(ep_hbm.at[0], ebuf.at[s], esem.at[s]).wait()
        pltpu.make_async_copy(nf_hbm.at[0], nbuf.at[s], esem.at[s]).wait()

    def _gather(ci_unused, s):
        pltpu.async_copy(h_hbm.at[ebuf.at[s, 0]], rows.at[s], gsem.at[s])

    def _wait_gather(s):
        pltpu.make_async_copy(h_hbm.at[ebuf.at[0, 0]], rows.at[s], gsem.at[s]).wait()

    def _scatter(s):
        pltpu.async_copy(rows.at[s], acc.at[ebuf.at[s, 1]], ssem.at[s], add=True)

    def _wait_scatter(s):
        pltpu.make_async_copy(rows.at[s], acc.at[ebuf.at[0, 1]], ssem.at[s]).wait()

    def _zb(i, _):
        r = i // 4
        q = i - r * 4
        zbuf[r, pl.ds(q * 16, 16)] = jnp.zeros((16,), jnp.float32)
        return 0
    lax.fori_loop(0, ZROWS * 4, _zb, 0)
    for j in range(-(-NZC // NS)):
        idx = sid + j * NS
        @pl.when(idx < NZC)
        def _():
            pltpu.sync_copy(zbuf, acc.at[pl.ds(idx * ZROWS, ZROWS)])
    plsc.subcore_barrier()

    _edge(0, 0)
    _edge(1, 1)
    _wait_edge(0)
    _gather(0, 0)

    def _iter(i, _):
        for s in range(RING):
            ci = RING * i + s
            s1 = (s + 1) % RING
            s2 = (s + 2) % RING
            @pl.when(ci + 2 < myn)
            def _():
                @pl.when(ci >= 2)
                def _():
                    _wait_scatter(s2)
                _edge(ci + 2, s2)
            @pl.when(ci + 1 < myn)
            def _():
                _wait_edge(s1)
                _gather(ci + 1, s1)
            _wait_gather(s)
            _mul_norm(nbuf, rows, s)
            _scatter(s)
        return 0
    lax.fori_loop(0, myn // RING, _iter, 0)
    for s in range(RING):
        _wait_scatter(s)
    plsc.subcore_barrier()

    for j in range(-(-NCD // NS)):
        idx = sid + j * NS
        @pl.when(idx < NCD)
        def _():
            r0 = idx * DRAIN
            pltpu.sync_copy(acc.at[pl.ds(r0, DRAIN)], out_hbm.at[cid, pl.ds(r0, DRAIN)])


@functools.cache
def _prop_kernel():
    return pl.kernel(
        _prop_body,
        out_type=jax.ShapeDtypeStruct((NC, BN, H), jnp.float32),
        mesh=plsc.VectorSubcoreMesh(core_axis_name="c", subcore_axis_name="s"),
        compiler_params=pltpu.CompilerParams(use_tc_tiling_on_sc=False),
        scratch_types=[
            pltpu.VMEM_SHARED((BN, H), jnp.float32),
            pltpu.VMEM((RING, 2, CHUNK), jnp.int32),
            pltpu.VMEM((RING, CHUNK), jnp.float32),
            pltpu.VMEM((RING, CHUNK, H), jnp.float32),
            pltpu.VMEM((ZROWS, H), jnp.float32),
            pltpu.SemaphoreType.DMA((RING,)),
            pltpu.SemaphoreType.DMA((RING,)),
            pltpu.SemaphoreType.DMA((RING,)),
        ],
    )


def _prop(h, epack, normf):
    return _prop_kernel()(h, epack, normf)


def _comb_a_body(p_ref, x_ref, w0_ref, w1_ref, dv_ref, part_ref, t1p_ref):
    dv = dv_ref[...]
    t1 = dv * (p_ref[0] + p_ref[1])
    t1p_ref[...] = dv * t1
    part_ref[...] = (jnp.dot(x_ref[...], w0_ref[...], preferred_element_type=jnp.float32)
                     + jnp.dot(t1, w1_ref[...], preferred_element_type=jnp.float32))


def _comb_a(p, x, w0T, w1T, dv):
    nb = BN // RBLK
    return pl.pallas_call(
        _comb_a_body,
        grid=(nb,),
        in_specs=[
            pl.BlockSpec((NC, RBLK, H), lambda i: (0, i, 0)),
            pl.BlockSpec((RBLK, H), lambda i: (i, 0)),
            pl.BlockSpec((H, H), lambda i: (0, 0)),
            pl.BlockSpec((H, H), lambda i: (0, 0)),
            pl.BlockSpec((RBLK, 1), lambda i: (i, 0)),
        ],
        out_specs=[
            pl.BlockSpec((RBLK, H), lambda i: (i, 0)),
            pl.BlockSpec((RBLK, H), lambda i: (i, 0)),
        ],
        out_shape=[
            jax.ShapeDtypeStruct((BN, H), jnp.float32),
            jax.ShapeDtypeStruct((BN, H), jnp.float32),
        ],
    )(p, x, w0T, w1T, dv)


def _comb_b_body(final, part_ref, p_ref, x_ref, w2_ref, b_ref, wo_ref, bo_ref,
                 dv_ref, y_ref, yp_ref):
    dv = dv_ref[...]
    t2 = 2.0 * dv * (p_ref[0] + p_ref[1]) - x_ref[...]
    y = (part_ref[...]
         + jnp.dot(t2, w2_ref[...], preferred_element_type=jnp.float32)
         + b_ref[...])
    if final:
        y_ref[...] = y
        yp_ref[...] = (jnp.dot(y, wo_ref[...], preferred_element_type=jnp.float32)
                       + bo_ref[...])
    else:
        y = jnp.maximum(y, 0.0)
        y_ref[...] = y
        yp_ref[...] = dv * y


def _comb_b(part, p, x, w2T, bvec, woT, bo, dv, final):
    nb = BN // RBLK
    body = functools.partial(_comb_b_body, final)
    out_specs = [pl.BlockSpec((RBLK, H), lambda i: (i, 0))]
    out_shape = [jax.ShapeDtypeStruct((BN, H), jnp.float32)]
    if final:
        out_specs.append(pl.BlockSpec((RBLK, 1), lambda i: (i, 0)))
        out_shape.append(jax.ShapeDtypeStruct((BN, 1), jnp.float32))
    else:
        out_specs.append(pl.BlockSpec((RBLK, H), lambda i: (i, 0)))
        out_shape.append(jax.ShapeDtypeStruct((BN, H), jnp.float32))
    return pl.pallas_call(
        body,
        grid=(nb,),
        in_specs=[
            pl.BlockSpec((RBLK, H), lambda i: (i, 0)),
            pl.BlockSpec((NC, RBLK, H), lambda i: (0, i, 0)),
            pl.BlockSpec((RBLK, H), lambda i: (i, 0)),
            pl.BlockSpec((H, H), lambda i: (0, 0)),
            pl.BlockSpec((1, H), lambda i: (0, 0)),
            pl.BlockSpec((H, 1), lambda i: (0, 0)),
            pl.BlockSpec((1, 1), lambda i: (0, 0)),
            pl.BlockSpec((RBLK, 1), lambda i: (i, 0)),
        ],
        out_specs=out_specs,
        out_shape=out_shape,
    )(part, p, x, w2T, bvec, woT, bo, dv)


def kernel(dynamic_features, static_features, edge_index, edge_weight,
           W_ih, W_hh, b_ih, b_hh, W_s, b_s, W_f, b_f,
           cheb1_W, cheb1_b, cheb2_W, cheb2_b, W_o, b_o):
    wihT = W_ih.T
    whhT = W_hh.T
    bg = (b_ih + b_hh).reshape(1, 4 * H)
    wsT = W_s.T
    bs = b_s.reshape(1, H)
    wfT = W_f.T
    bf = b_f.reshape(1, H)
    c1 = [cheb1_W[k].T for k in range(3)]
    c2 = [cheb2_W[k].T for k in range(3)]
    b1 = cheb1_b.reshape(1, H)
    b2 = cheb2_b.reshape(1, H)
    woT = W_o.T
    bo = b_o.reshape(1, 1)

    src0 = edge_index[0]
    dst0 = edge_index[1]

    deg = jnp.zeros((N,), jnp.float32).at[src0].add(edge_weight)
    dinv = jnp.where(deg > 0, lax.rsqrt(jnp.where(deg > 0, deg, 1.0)), 0.0)
    dv = jnp.concatenate([dinv, dinv]).reshape(BN, 1)

    pad = E_PAD - B * E
    src_full = jnp.concatenate([src0, src0 + N, jnp.zeros((pad,), jnp.int32)])
    dst_full = jnp.concatenate([dst0, dst0 + N, jnp.zeros((pad,), jnp.int32)])
    ew_full = jnp.concatenate([-edge_weight, -edge_weight,
                               jnp.zeros((pad,), jnp.float32)])
    epack = jnp.stack([
        src_full.reshape(NCHT, CHUNK),
        dst_full.reshape(NCHT, CHUNK),
    ], axis=1)
    normf = ew_full.reshape(NCHT, CHUNK)

    x0, x0p = _encoder(dynamic_features, static_features,
                       wihT, whhT, bg, wsT, bs, wfT, bf, dv)

    p1 = _prop(x0p, epack, normf)
    part1, t1p = _comb_a(p1, x0, c1[0], c1[1], dv)
    p2 = _prop(t1p, epack, normf)
    y1, y1p = _comb_b(part1, p2, x0, c1[2], b1, woT, bo, dv, final=False)

    q1 = _prop(y1p, epack, normf)
    part2, u1p = _comb_a(q1, y1, c2[0], c2[1], dv)
    q2 = _prop(u1p, epack, normf)
    _, pred = _comb_b(part2, q2, y1, c2[2], b2, woT, bo, dv, final=True)

    return pred.reshape(B, N)


# speedup vs baseline: 13.3269x; 1.1098x over previous
import functools

import jax
import jax.numpy as jnp
from jax import lax
from jax.experimental import pallas as pl
from jax.experimental.pallas import tpu as pltpu
from jax.experimental.pallas import tpu_sc as plsc

B, T, N, F_DYN = 2, 12, 10000, 8
F_STA = 16
H = 64
BN = B * N
E = 320000

NC, NS = 2, 16
NW = NC * NS
CHUNK = 128
RING = 4
NCHUNK = RING * (-(-(B * E) // (NW * CHUNK * RING)))
CA = 212
CB = 2 * NCHUNK - CA
EPW = NCHUNK * CHUNK
E_PAD = EPW * NW
NCHT = E_PAD // CHUNK

RBLK = 2000
DRAIN = 400
NCD = BN // DRAIN
ZROWS = 80
NZC = BN // ZROWS


def _encoder_body(x_ref, sta_ref, wih_ref, whh_ref, bg_ref, ws_ref, bs_ref,
                  wf_ref, bf_ref, out_ref):
    wih = wih_ref[...]
    whh = whh_ref[...]
    bg = bg_ref[...]
    h = jnp.zeros((RBLK, H), jnp.float32)
    c = jnp.zeros((RBLK, H), jnp.float32)
    for t in range(T):
        xt = x_ref[0, t]
        g = (jnp.dot(xt, wih, preferred_element_type=jnp.float32)
             + jnp.dot(h, whh, preferred_element_type=jnp.float32) + bg)
        i_g = jax.nn.sigmoid(g[:, 0:H])
        f_g = jax.nn.sigmoid(g[:, H:2 * H])
        g_g = jnp.tanh(g[:, 2 * H:3 * H])
        o_g = jax.nn.sigmoid(g[:, 3 * H:4 * H])
        c = f_g * c + i_g * g_g
        h = o_g * jnp.tanh(c)
    s = jnp.maximum(
        jnp.dot(sta_ref[0], ws_ref[...], preferred_element_type=jnp.float32)
        + bs_ref[...], 0.0)
    wf = wf_ref[...]
    fused = (jnp.dot(h, wf[0:H], preferred_element_type=jnp.float32)
             + jnp.dot(s, wf[H:2 * H], preferred_element_type=jnp.float32)
             + bf_ref[...])
    out_ref[...] = jnp.maximum(fused, 0.0)


def _encoder(dyn, sta, wihT, whhT, bg, wsT, bs, wfT, bf):
    nb = N // RBLK
    return pl.pallas_call(
        _encoder_body,
        grid=(B, nb),
        in_specs=[
            pl.BlockSpec((1, T, RBLK, F_DYN), lambda b, i: (b, 0, i, 0)),
            pl.BlockSpec((1, RBLK, F_STA), lambda b, i: (b, i, 0)),
            pl.BlockSpec((F_DYN, 4 * H), lambda b, i: (0, 0)),
            pl.BlockSpec((H, 4 * H), lambda b, i: (0, 0)),
            pl.BlockSpec((1, 4 * H), lambda b, i: (0, 0)),
            pl.BlockSpec((F_STA, H), lambda b, i: (0, 0)),
            pl.BlockSpec((1, H), lambda b, i: (0, 0)),
            pl.BlockSpec((2 * H, H), lambda b, i: (0, 0)),
            pl.BlockSpec((1, H), lambda b, i: (0, 0)),
        ],
        out_specs=pl.BlockSpec((RBLK, H), lambda b, i: (b * nb + i, 0)),
        out_shape=jax.ShapeDtypeStruct((BN, H), jnp.float32),
    )(dyn, sta, wihT, whhT, bg, wsT, bs, wfT, bf)


def _mul_norm(nbuf, rows, s):
    def _group(g, _):
        nvec = nbuf[s, pl.ds(g * 16, 16)]
        for l in range(16):
            e = g * 16 + l
            nsp = jnp.broadcast_to(nvec[l], (16,))
            for q in range(H // 16):
                rows[s, e, pl.ds(q * 16, 16)] = rows[s, e, pl.ds(q * 16, 16)] * nsp
        return 0
    lax.fori_loop(0, CHUNK // 16, _group, 0)


def _prop_body(h_hbm, ep_hbm, nf_hbm, out_hbm, acc, ebuf, nbuf, rows, zbuf,
               esem, gsem, ssem):
    cid = lax.axis_index("c")
    sid = lax.axis_index("s")
    myn = jnp.where(cid == 0, CA, CB)
    base = jnp.where(cid == 0, sid * CA, NS * CA + sid * CB)

    def _edge(ci, s):
        pltpu.async_copy(ep_hbm.at[base + ci], ebuf.at[s], esem.at[s])
        pltpu.async_copy(nf_hbm.at[base + ci], nbuf.at[s], esem.at[s])

    def _wait_edge(s):
        pltpu.make_async_copy(ep_hbm.at[0], ebuf.at[s], esem.at[s]).wait()
        pltpu.make_async_copy(nf_hbm.at[0], nbuf.at[s], esem.at[s]).wait()

    def _gather(ci_unused, s):
        pltpu.async_copy(h_hbm.at[ebuf.at[s, 0]], rows.at[s], gsem.at[s])

    def _wait_gather(s):
        pltpu.make_async_copy(h_hbm.at[ebuf.at[0, 0]], rows.at[s], gsem.at[s]).wait()

    def _scatter(s):
        pltpu.async_copy(rows.at[s], acc.at[ebuf.at[s, 1]], ssem.at[s], add=True)

    def _wait_scatter(s):
        pltpu.make_async_copy(rows.at[s], acc.at[ebuf.at[0, 1]], ssem.at[s]).wait()

    def _zb(i, _):
        r = i // 4
        q = i - r * 4
        zbuf[r, pl.ds(q * 16, 16)] = jnp.zeros((16,), jnp.float32)
        return 0
    lax.fori_loop(0, ZROWS * 4, _zb, 0)
    for j in range(-(-NZC // NS)):
        idx = sid + j * NS
        @pl.when(idx < NZC)
        def _():
            pltpu.sync_copy(zbuf, acc.at[pl.ds(idx * ZROWS, ZROWS)])
    plsc.subcore_barrier()

    _edge(0, 0)
    _edge(1, 1)
    _wait_edge(0)
    _gather(0, 0)

    def _iter(i, _):
        for s in range(RING):
            ci = RING * i + s
            s1 = (s + 1) % RING
            s2 = (s + 2) % RING
            @pl.when(ci + 2 < myn)
            def _():
                @pl.when(ci >= 2)
                def _():
                    _wait_scatter(s2)
                _edge(ci + 2, s2)
            @pl.when(ci + 1 < myn)
            def _():
                _wait_edge(s1)
                _gather(ci + 1, s1)
            _wait_gather(s)
            _mul_norm(nbuf, rows, s)
            _scatter(s)
        return 0
    lax.fori_loop(0, myn // RING, _iter, 0)
    for s in range(RING):
        _wait_scatter(s)
    plsc.subcore_barrier()

    for j in range(-(-NCD // NS)):
        idx = sid + j * NS
        @pl.when(idx < NCD)
        def _():
            r0 = idx * DRAIN
            pltpu.sync_copy(acc.at[pl.ds(r0, DRAIN)], out_hbm.at[cid, pl.ds(r0, DRAIN)])


@functools.cache
def _prop_kernel():
    return pl.kernel(
        _prop_body,
        out_type=jax.ShapeDtypeStruct((NC, BN, H), jnp.float32),
        mesh=plsc.VectorSubcoreMesh(core_axis_name="c", subcore_axis_name="s"),
        compiler_params=pltpu.CompilerParams(use_tc_tiling_on_sc=False),
        scratch_types=[
            pltpu.VMEM_SHARED((BN, H), jnp.float32),
            pltpu.VMEM((RING, 2, CHUNK), jnp.int32),
            pltpu.VMEM((RING, CHUNK), jnp.float32),
            pltpu.VMEM((RING, CHUNK, H), jnp.float32),
            pltpu.VMEM((ZROWS, H), jnp.float32),
            pltpu.SemaphoreType.DMA((RING,)),
            pltpu.SemaphoreType.DMA((RING,)),
            pltpu.SemaphoreType.DMA((RING,)),
        ],
    )


def _prop(h, epack, normf):
    return _prop_kernel()(h, epack, normf)


NDCH = E // CHUNK
NDW = -(-NDCH // NW)


def _deg_body(ep_hbm, nf_hbm, out_hbm, dacc, ebuf, nbuf, zbuf, esem, ssem):
    cid = lax.axis_index("c")
    sid = lax.axis_index("s")
    wid = sid * NC + cid
    lo = wid * NDW
    hi = jnp.minimum(lo + NDW, NDCH)

    def _edge(ci, s):
        pltpu.async_copy(ep_hbm.at[ci], ebuf.at[s], esem.at[s])
        pltpu.async_copy(nf_hbm.at[ci], nbuf.at[s], esem.at[s])

    def _wait_edge(s):
        pltpu.make_async_copy(ep_hbm.at[0], ebuf.at[s], esem.at[s]).wait()
        pltpu.make_async_copy(nf_hbm.at[0], nbuf.at[s], esem.at[s]).wait()

    def _scat(s):
        pltpu.async_copy(nbuf.at[s], dacc.at[ebuf.at[s, 0]], ssem.at[s], add=True)

    def _wait_scat(s):
        pltpu.make_async_copy(nbuf.at[s], dacc.at[ebuf.at[0, 0]], ssem.at[s]).wait()

    def _zb(i, _):
        zbuf[pl.ds(i * 16, 16)] = jnp.zeros((16,), jnp.float32)
        return 0
    lax.fori_loop(0, 25, _zb, 0)
    for j in range(2):
        idx = sid + j * NS
        @pl.when(idx < N // 400)
        def _():
            pltpu.sync_copy(zbuf, dacc.at[pl.ds(idx * 400, 400)])
    plsc.subcore_barrier()

    @pl.when(lo < hi)
    def _():
        _edge(lo, 0)

        def _it(i, _):
            for s in (0, 1):
                ci = lo + 2 * i + s
                o = 1 - s
                @pl.when(ci < hi)
                def _():
                    @pl.when(ci + 1 < hi)
                    def _():
                        @pl.when(ci >= lo + 1)
                        def _():
                            _wait_scat(o)
                        _edge(ci + 1, o)
                    _wait_edge(s)
                    _scat(s)
            return 0
        lax.fori_loop(0, (NDW + 1) // 2, _it, 0)
        _wait_scat(0)
        @pl.when(hi - lo >= 2)
        def _():
            _wait_scat(1)
    plsc.subcore_barrier()

    for j in range(2):
        idx = sid + j * NS
        @pl.when(idx < N // 400)
        def _():
            pltpu.sync_copy(dacc.at[pl.ds(idx * 400, 400)],
                            out_hbm.at[cid, pl.ds(idx * 400, 400)])


@functools.cache
def _deg_kernel():
    return pl.kernel(
        _deg_body,
        out_type=jax.ShapeDtypeStruct((NC, N), jnp.float32),
        mesh=plsc.VectorSubcoreMesh(core_axis_name="c", subcore_axis_name="s"),
        compiler_params=pltpu.CompilerParams(use_tc_tiling_on_sc=False),
        scratch_types=[
            pltpu.VMEM_SHARED((N,), jnp.float32),
            pltpu.VMEM((2, 2, CHUNK), jnp.int32),
            pltpu.VMEM((2, CHUNK), jnp.float32),
            pltpu.VMEM((400,), jnp.float32),
            pltpu.SemaphoreType.DMA((2,)),
            pltpu.SemaphoreType.DMA((2,)),
        ],
    )


def _prescale_body(x_ref, d_ref, xp_ref, dv_ref):
    deg = -(d_ref[0] + d_ref[1])
    dv = jnp.where(deg > 0, lax.rsqrt(jnp.where(deg > 0, deg, 1.0)), 0.0)
    dv_ref[...] = dv
    xp_ref[...] = dv * x_ref[...]


def _prescale(x0, dpart):
    nb = BN // RBLK
    nbh = N // RBLK
    return pl.pallas_call(
        _prescale_body,
        grid=(nb,),
        in_specs=[
            pl.BlockSpec((RBLK, H), lambda i: (i, 0)),
            pl.BlockSpec((NC, RBLK, 1), lambda i: (0, i % nbh, 0)),
        ],
        out_specs=[
            pl.BlockSpec((RBLK, H), lambda i: (i, 0)),
            pl.BlockSpec((RBLK, 1), lambda i: (i, 0)),
        ],
        out_shape=[
            jax.ShapeDtypeStruct((BN, H), jnp.float32),
            jax.ShapeDtypeStruct((BN, 1), jnp.float32),
        ],
    )(x0, dpart)


def _comb_a_body(p_ref, x_ref, w0_ref, w1_ref, dv_ref, part_ref, t1p_ref):
    dv = dv_ref[...]
    t1 = dv * (p_ref[0] + p_ref[1])
    t1p_ref[...] = dv * t1
    part_ref[...] = (jnp.dot(x_ref[...], w0_ref[...], preferred_element_type=jnp.float32)
                     + jnp.dot(t1, w1_ref[...], preferred_element_type=jnp.float32))


def _comb_a(p, x, w0T, w1T, dv):
    nb = BN // RBLK
    return pl.pallas_call(
        _comb_a_body,
        grid=(nb,),
        in_specs=[
            pl.BlockSpec((NC, RBLK, H), lambda i: (0, i, 0)),
            pl.BlockSpec((RBLK, H), lambda i: (i, 0)),
            pl.BlockSpec((H, H), lambda i: (0, 0)),
            pl.BlockSpec((H, H), lambda i: (0, 0)),
            pl.BlockSpec((RBLK, 1), lambda i: (i, 0)),
        ],
        out_specs=[
            pl.BlockSpec((RBLK, H), lambda i: (i, 0)),
            pl.BlockSpec((RBLK, H), lambda i: (i, 0)),
        ],
        out_shape=[
            jax.ShapeDtypeStruct((BN, H), jnp.float32),
            jax.ShapeDtypeStruct((BN, H), jnp.float32),
        ],
    )(p, x, w0T, w1T, dv)


def _comb_b_body(final, part_ref, p_ref, x_ref, w2_ref, b_ref, wo_ref, bo_ref,
                 dv_ref, y_ref, yp_ref):
    dv = dv_ref[...]
    t2 = 2.0 * dv * (p_ref[0] + p_ref[1]) - x_ref[...]
    y = (part_ref[...]
         + jnp.dot(t2, w2_ref[...], preferred_element_type=jnp.float32)
         + b_ref[...])
    if final:
        y_ref[...] = y
        yp_ref[...] = (jnp.dot(y, wo_ref[...], preferred_element_type=jnp.float32)
                       + bo_ref[...])
    else:
        y = jnp.maximum(y, 0.0)
        y_ref[...] = y
        yp_ref[...] = dv * y


def _comb_b(part, p, x, w2T, bvec, woT, bo, dv, final):
    nb = BN // RBLK
    body = functools.partial(_comb_b_body, final)
    out_specs = [pl.BlockSpec((RBLK, H), lambda i: (i, 0))]
    out_shape = [jax.ShapeDtypeStruct((BN, H), jnp.float32)]
    if final:
        out_specs.append(pl.BlockSpec((RBLK, 1), lambda i: (i, 0)))
        out_shape.append(jax.ShapeDtypeStruct((BN, 1), jnp.float32))
    else:
        out_specs.append(pl.BlockSpec((RBLK, H), lambda i: (i, 0)))
        out_shape.append(jax.ShapeDtypeStruct((BN, H), jnp.float32))
    return pl.pallas_call(
        body,
        grid=(nb,),
        in_specs=[
            pl.BlockSpec((RBLK, H), lambda i: (i, 0)),
            pl.BlockSpec((NC, RBLK, H), lambda i: (0, i, 0)),
            pl.BlockSpec((RBLK, H), lambda i: (i, 0)),
            pl.BlockSpec((H, H), lambda i: (0, 0)),
            pl.BlockSpec((1, H), lambda i: (0, 0)),
            pl.BlockSpec((H, 1), lambda i: (0, 0)),
            pl.BlockSpec((1, 1), lambda i: (0, 0)),
            pl.BlockSpec((RBLK, 1), lambda i: (i, 0)),
        ],
        out_specs=out_specs,
        out_shape=out_shape,
    )(part, p, x, w2T, bvec, woT, bo, dv)


def kernel(dynamic_features, static_features, edge_index, edge_weight,
           W_ih, W_hh, b_ih, b_hh, W_s, b_s, W_f, b_f,
           cheb1_W, cheb1_b, cheb2_W, cheb2_b, W_o, b_o):
    wihT = W_ih.T
    whhT = W_hh.T
    bg = (b_ih + b_hh).reshape(1, 4 * H)
    wsT = W_s.T
    bs = b_s.reshape(1, H)
    wfT = W_f.T
    bf = b_f.reshape(1, H)
    c1 = [cheb1_W[k].T for k in range(3)]
    c2 = [cheb2_W[k].T for k in range(3)]
    b1 = cheb1_b.reshape(1, H)
    b2 = cheb2_b.reshape(1, H)
    woT = W_o.T
    bo = b_o.reshape(1, 1)

    src0 = edge_index[0]
    dst0 = edge_index[1]

    pad = E_PAD - B * E
    src_full = jnp.concatenate([src0, src0 + N, jnp.zeros((pad,), jnp.int32)])
    dst_full = jnp.concatenate([dst0, dst0 + N, jnp.zeros((pad,), jnp.int32)])
    ew_full = jnp.concatenate([-edge_weight, -edge_weight,
                               jnp.zeros((pad,), jnp.float32)])
    epack = jnp.stack([
        src_full.reshape(NCHT, CHUNK),
        dst_full.reshape(NCHT, CHUNK),
    ], axis=1)
    normf = ew_full.reshape(NCHT, CHUNK)

    dpart = _deg_kernel()(epack, normf)
    x0 = _encoder(dynamic_features, static_features,
                  wihT, whhT, bg, wsT, bs, wfT, bf)
    x0p, dv = _prescale(x0, dpart.reshape(NC, N, 1))

    p1 = _prop(x0p, epack, normf)
    part1, t1p = _comb_a(p1, x0, c1[0], c1[1], dv)
    p2 = _prop(t1p, epack, normf)
    y1, y1p = _comb_b(part1, p2, x0, c1[2], b1, woT, bo, dv, final=False)

    q1 = _prop(y1p, epack, normf)
    part2, u1p = _comb_a(q1, y1, c2[0], c2[1], dv)
    q2 = _prop(u1p, epack, normf)
    _, pred = _comb_b(part2, q2, y1, c2[2], b2, woT, bo, dv, final=True)

    return pred.reshape(B, N)
